# baseline (speedup 1.0000x reference)
import functools

import jax
import jax.numpy as jnp
from jax import lax
from jax.experimental import pallas as pl
from jax.experimental.pallas import tpu as pltpu
from jax.experimental.pallas import tpu_sc as plsc

_NUM_WORKERS = 32
_LANES = 16


def kernel(x, row_embed, col_embed):
    b, _, h, w = x.shape
    n_rows, d = col_embed.shape
    c_total = 2 * d
    ch_per_w = c_total // _NUM_WORKERS
    hw = h * w
    blk_words = ch_per_w * hw

    mesh = plsc.VectorSubcoreMesh(core_axis_name="c", subcore_axis_name="s")

    @functools.partial(
        pl.kernel,
        mesh=mesh,
        out_type=jax.ShapeDtypeStruct((b, c_total * hw), jnp.float32),
        scratch_types=[
            pltpu.VMEM((n_rows * d,), jnp.float32),
            pltpu.VMEM((n_rows * d,), jnp.float32),
            pltpu.VMEM((blk_words,), jnp.float32),
            pltpu.SemaphoreType.DMA,
        ],
        compiler_params=pltpu.CompilerParams(needs_layout_passes=False),
    )
    def pos_kernel(row_hbm, col_hbm, out_hbm, row_v, col_v, blk, sem):
        wid = lax.axis_index("s") * 2 + lax.axis_index("c")
        c0 = wid * ch_per_w

        pltpu.sync_copy(row_hbm, row_v)
        pltpu.sync_copy(col_hbm, col_v)

        lanes = lax.iota(jnp.int32, _LANES)

        @pl.when(wid < _NUM_WORKERS // 2)
        def _():
            def ch_body(j, carry):
                cvec = jnp.full((_LANES,), c0 + j, jnp.int32)
                v0 = plsc.load_gather(col_v, [lanes * d + cvec])
                v1 = plsc.load_gather(col_v, [(lanes + _LANES) * d + cvec])

                def h_body(hh, c2):
                    base = j * hw + hh * w
                    blk[pl.ds(base, _LANES)] = v0
                    blk[pl.ds(base + _LANES, _LANES)] = v1
                    return c2

                return lax.fori_loop(0, h, h_body, carry)

            lax.fori_loop(0, ch_per_w, ch_body, 0)

        @pl.when(wid >= _NUM_WORKERS // 2)
        def _():
            def ch_body(j, carry):
                ccvec = jnp.full((_LANES,), c0 - d + j, jnp.int32)

                def h_body(hh, c2):
                    v = plsc.load_gather(row_v, [ccvec + hh * d])
                    base = j * hw + hh * w
                    blk[pl.ds(base, _LANES)] = v
                    blk[pl.ds(base + _LANES, _LANES)] = v
                    return c2

                return lax.fori_loop(0, h, h_body, carry)

            lax.fori_loop(0, ch_per_w, ch_body, 0)

        copies = [
            pltpu.async_copy(blk, out_hbm.at[bb, pl.ds(c0 * hw, blk_words)], sem)
            for bb in range(b)
        ]
        for cp in copies:
            cp.wait()

    out = pos_kernel(row_embed.reshape(-1), col_embed.reshape(-1))
    return out.reshape(b, c_total, h, w)


# speedup vs baseline: 1.4228x; 1.4228x over previous
import functools

import jax
import jax.numpy as jnp
from jax import lax
from jax.experimental import pallas as pl
from jax.experimental.pallas import tpu as pltpu
from jax.experimental.pallas import tpu_sc as plsc

_NUM_WORKERS = 32
_LANES = 16


def kernel(x, row_embed, col_embed):
    b, _, h, w = x.shape
    n_rows, d = col_embed.shape
    c_total = 2 * d
    ch_per_w = c_total // _NUM_WORKERS
    hw = h * w

    mesh = plsc.VectorSubcoreMesh(core_axis_name="c", subcore_axis_name="s")

    @functools.partial(
        pl.kernel,
        mesh=mesh,
        out_type=jax.ShapeDtypeStruct((b, c_total, h, w), jnp.float32),
        scratch_types=[
            pltpu.VMEM((n_rows * d,), jnp.float32),
            pltpu.VMEM((ch_per_w, h, w), jnp.float32),
            pltpu.SemaphoreType.DMA,
        ],
        compiler_params=pltpu.CompilerParams(needs_layout_passes=False),
    )
    def pos_kernel(row_hbm, col_hbm, out_hbm, tab_v, blk, sem):
        wid = lax.axis_index("s") * 2 + lax.axis_index("c")
        c0 = wid * ch_per_w
        is_col = wid < _NUM_WORKERS // 2

        lanes = lax.iota(jnp.int32, _LANES)

        @pl.when(is_col)
        def _():
            pltpu.sync_copy(col_hbm, tab_v)

            def ch_body(j, carry):
                cvec = jnp.full((_LANES,), c0 + j, jnp.int32)
                v0 = plsc.load_gather(tab_v, [lanes * d + cvec])
                v1 = plsc.load_gather(tab_v, [(lanes + _LANES) * d + cvec])

                def h_body(hh, c2):
                    blk[j, hh, pl.ds(0, _LANES)] = v0
                    blk[j, hh, pl.ds(_LANES, _LANES)] = v1
                    return c2

                return lax.fori_loop(0, h, h_body, carry)

            lax.fori_loop(0, ch_per_w, ch_body, 0)

        @pl.when(jnp.logical_not(is_col))
        def _():
            pltpu.sync_copy(row_hbm, tab_v)

            def ch_body(j, carry):
                ccvec = jnp.full((_LANES,), c0 - d + j, jnp.int32)

                def h_body(hh, c2):
                    v = plsc.load_gather(tab_v, [ccvec + hh * d])
                    blk[j, hh, pl.ds(0, _LANES)] = v
                    blk[j, hh, pl.ds(_LANES, _LANES)] = v
                    return c2

                return lax.fori_loop(0, h, h_body, carry)

            lax.fori_loop(0, ch_per_w, ch_body, 0)

        copies = [
            pltpu.async_copy(blk, out_hbm.at[bb, pl.ds(c0, ch_per_w)], sem)
            for bb in range(b)
        ]
        for cp in copies:
            cp.wait()

    return pos_kernel(row_embed.reshape(-1), col_embed.reshape(-1))


# speedup vs baseline: 6.3908x; 4.4916x over previous
import functools

import jax
import jax.numpy as jnp
from jax import lax
from jax.experimental import pallas as pl
from jax.experimental.pallas import tpu as pltpu
from jax.experimental.pallas import tpu_sc as plsc

_NUM_WORKERS = 32
_LANES = 16


def kernel(x, row_embed, col_embed):
    b, _, h, w = x.shape
    n_rows, d = col_embed.shape
    c_total = 2 * d

    mesh = plsc.VectorSubcoreMesh(core_axis_name="c", subcore_axis_name="s")

    @functools.partial(
        pl.kernel,
        mesh=mesh,
        out_type=jax.ShapeDtypeStruct((b, h, w, c_total), jnp.float32),
        scratch_types=[
            pltpu.VMEM((d,), jnp.float32),
            pltpu.VMEM((w, c_total), jnp.float32),
            pltpu.SemaphoreType.DMA,
        ],
        compiler_params=pltpu.CompilerParams(needs_layout_passes=False),
    )
    def pos_kernel(row_hbm, col_hbm, out_hbm, row_v, blk, sem):
        wid = lax.axis_index("s") * 2 + lax.axis_index("c")
        hh = wid

        pltpu.sync_copy(col_hbm.at[pl.ds(0, w), :], blk.at[:, pl.ds(0, d)])
        pltpu.sync_copy(row_hbm.at[hh], row_v)

        segs = [row_v[pl.ds(k * _LANES, _LANES)] for k in range(d // _LANES)]

        def w_body(ww, carry):
            for k, v in enumerate(segs):
                blk[ww, pl.ds(d + k * _LANES, _LANES)] = v
            return carry

        lax.fori_loop(0, w, w_body, 0)

        copies = [
            pltpu.async_copy(blk, out_hbm.at[bb, hh], sem) for bb in range(b)
        ]
        for cp in copies:
            cp.wait()

    out = pos_kernel(row_embed, col_embed)
    return jnp.transpose(out, (0, 3, 1, 2))
